# Initial kernel scaffold; baseline (speedup 1.0000x reference)
import jax
import jax.numpy as jnp
from jax.experimental import pallas as pl


def kernel(categorical, continuous, binary, emb, W1, b1, g1, be1, W2, b2, g2, be2, W3, b3):
    raise NotImplementedError("write your pallas kernel here")


# speedup vs baseline: 7.8667x; 7.8667x over previous
import functools

import jax
import jax.numpy as jnp
from jax import lax
from jax.experimental import pallas as pl
from jax.experimental.pallas import tpu as pltpu
from jax.experimental.pallas import tpu_sc as plsc

B = 16384
NCAT = 26
V = 100000
D = 16
NCONT = 13
NBIN = 16
H1 = 128
H2 = 64
H2P = 128
CBW = 32
EMBW = NCAT * D

try:
    _info = plsc.get_sparse_core_info()
    _NC = _info.num_cores
    _NS = _info.num_subcores
except Exception:
    _NC, _NS = 2, 16
NW = _NC * _NS
ROWS = B * NCAT
ROWS_PER_W = ROWS // NW
N_CHUNKS = 8
CH = ROWS_PER_W // N_CHUNKS


def _sc_gather(table, idx):
    mesh = plsc.VectorSubcoreMesh(core_axis_name="c", subcore_axis_name="s")

    @functools.partial(
        pl.kernel,
        mesh=mesh,
        compiler_params=pltpu.CompilerParams(use_tc_tiling_on_sc=False),
        out_type=jax.ShapeDtypeStruct((ROWS, D), jnp.float32),
        scratch_types=[
            pltpu.VMEM((ROWS_PER_W,), jnp.int32),
            pltpu.VMEM((CH, D), jnp.float32),
            pltpu.VMEM((CH, D), jnp.float32),
            pltpu.SemaphoreType.DMA,
            pltpu.SemaphoreType.DMA,
        ],
    )
    def gather_k(table_hbm, idx_hbm, out_hbm, idx_v, buf0, buf1, sem0, sem1):
        wid = lax.axis_index("s") * _NC + lax.axis_index("c")
        base = wid * ROWS_PER_W
        pltpu.sync_copy(idx_hbm.at[pl.ds(base, ROWS_PER_W)], idx_v)
        bufs = (buf0, buf1)
        sems = (sem0, sem1)

        def issue(c):
            return pltpu.async_copy(
                table_hbm.at[idx_v.at[pl.ds(c * CH, CH)]], bufs[c % 2], sems[c % 2]
            )

        cps = [None, None]
        cps[0] = issue(0)
        for c in range(N_CHUNKS):
            if c + 1 < N_CHUNKS:
                cps[(c + 1) % 2] = issue(c + 1)
            cps[c % 2].wait()
            pltpu.sync_copy(bufs[c % 2], out_hbm.at[pl.ds(base + c * CH, CH)])

    return gather_k(table, idx)


BT = 2048


def _mlp_body(xg_ref, cb_ref, w1e_ref, w1cb_ref, b1_ref, w2_ref, b2_ref,
              w3_ref, out_ref):
    h = jnp.dot(xg_ref[...], w1e_ref[...], preferred_element_type=jnp.float32)
    h = h + jnp.dot(cb_ref[...], w1cb_ref[...],
                    preferred_element_type=jnp.float32)
    h = jnp.maximum(h + b1_ref[...], 0.0)
    h2 = jnp.dot(h, w2_ref[...], preferred_element_type=jnp.float32)
    h2 = jnp.maximum(h2 + b2_ref[...], 0.0)
    out_ref[...] = jnp.sum(h2 * w3_ref[...], axis=1)


def _tc_mlp(xg, cb, w1e, w1cb, b1f, w2f, b2f, w3f):
    grid = (B // BT,)
    return pl.pallas_call(
        _mlp_body,
        grid=grid,
        in_specs=[
            pl.BlockSpec((BT, EMBW), lambda i: (i, 0)),
            pl.BlockSpec((BT, CBW), lambda i: (i, 0)),
            pl.BlockSpec((EMBW, H1), lambda i: (0, 0)),
            pl.BlockSpec((CBW, H1), lambda i: (0, 0)),
            pl.BlockSpec((1, H1), lambda i: (0, 0)),
            pl.BlockSpec((H1, H2P), lambda i: (0, 0)),
            pl.BlockSpec((1, H2P), lambda i: (0, 0)),
            pl.BlockSpec((1, H2P), lambda i: (0, 0)),
        ],
        out_specs=pl.BlockSpec((BT,), lambda i: (i,)),
        out_shape=jax.ShapeDtypeStruct((B,), jnp.float32),
    )(xg, cb, w1e, w1cb, b1f, w2f, b2f, w3f)


def kernel(categorical, continuous, binary, emb, W1, b1, g1, be1,
           W2, b2, g2, be2, W3, b3):
    eps = 1e-5
    inv = 1.0 / jnp.sqrt(1.0 + eps)
    s1 = g1 * inv
    w1f = W1 * s1[:, None]
    b1f = b1 * s1 + be1
    s2 = g2 * inv
    w2f = W2 * s2[:, None]
    b2f = b2 * s2 + be2

    ncb = NCONT + NBIN
    w1cb = jnp.zeros((CBW, H1), jnp.float32).at[:ncb].set(w1f[:, :ncb].T)
    w1e = w1f[:, ncb:].T
    cb = jnp.concatenate(
        [continuous, binary,
         jnp.zeros((B, CBW - ncb), jnp.float32)], axis=1)

    w2p = jnp.zeros((H1, H2P), jnp.float32).at[:, :H2].set(w2f.T)
    b2p = jnp.zeros((1, H2P), jnp.float32).at[0, :H2].set(b2f)
    w3p = jnp.zeros((1, H2P), jnp.float32).at[0, :H2].set(W3[0])

    idx = (categorical + (jnp.arange(NCAT, dtype=jnp.int32) * V)[None, :])
    idx = idx.reshape(-1)
    table = emb.reshape(NCAT * V, D)

    xg = _sc_gather(table, idx).reshape(B, EMBW)

    out = _tc_mlp(xg, cb, w1e, w1cb, b1f.reshape(1, H1), w2p, b2p, w3p)
    return out + b3[0]
